# Initial kernel scaffold; baseline (speedup 1.0000x reference)
import functools

import jax
import jax.numpy as jnp
from jax import lax
from jax.experimental import pallas as pl
from jax.experimental.pallas import tpu as pltpu
from jax.experimental.pallas import tpu_sc as plsc

B = 4096
N_DENSE = 13
N_TABLES = 26
VOCAB = 100000
EMB_DIM = 32
N_FEAT = N_TABLES + 1
N_INTERACT = N_FEAT * (N_FEAT - 1) // 2

_SC_NUM_CORES = 2
_SC_NUM_SUBCORES = 16
_NW = _SC_NUM_CORES * _SC_NUM_SUBCORES

_BB = 512


def _sc_gather(table_flat, idx_flat):
    ni = idx_flat.shape[0]
    b_per_w = ni // _NW
    mesh = plsc.VectorSubcoreMesh(core_axis_name="c", subcore_axis_name="s")

    @functools.partial(
        pl.kernel,
        mesh=mesh,
        out_type=jax.ShapeDtypeStruct((ni, EMB_DIM), jnp.float32),
        scratch_types=[
            pltpu.VMEM((b_per_w,), jnp.int32),
            pltpu.VMEM((b_per_w, EMB_DIM), jnp.float32),
            pltpu.SemaphoreType.DMA,
        ],
    )
    def gather_kernel(table_hbm, idx_hbm, out_hbm, idx_v, rows_v, sem):
        wid = lax.axis_index("s") * _SC_NUM_CORES + lax.axis_index("c")
        base = wid * b_per_w
        pltpu.sync_copy(idx_hbm.at[pl.ds(base, b_per_w)], idx_v)
        pltpu.async_copy(table_hbm.at[idx_v], rows_v, sem).wait()
        pltpu.sync_copy(rows_v, out_hbm.at[pl.ds(base, b_per_w)])

    return gather_kernel(table_flat, idx_flat)


def _tc_body(xt_ref, g_ref,
             bw0_ref, bb0_ref, bw1_ref, bb1_ref, bw2_ref, bb2_ref,
             tw0a_ref, tw0b_ref, tb0_ref, tw1_ref, tb1_ref,
             tw2_ref, tb2_ref, tw3_ref, tb3_ref, tw4_ref, tb4_ref,
             out_ref, et_ref, inter_ref):
    f32 = jnp.float32
    x = xt_ref[...]
    h = jnp.maximum(jnp.dot(bw0_ref[...], x, preferred_element_type=f32)
                    + bb0_ref[...], 0.0)
    h = jnp.maximum(jnp.dot(bw1_ref[...], h, preferred_element_type=f32)
                    + bb1_ref[...], 0.0)
    x32 = jnp.maximum(jnp.dot(bw2_ref[...], h, preferred_element_type=f32)
                      + bb2_ref[...], 0.0)

    et_ref[0:N_TABLES * EMB_DIM, :] = jnp.transpose(g_ref[...])
    et_ref[N_TABLES * EMB_DIM:, :] = x32

    row = 0
    for i in range(1, N_FEAT):
        ei = et_ref[i * EMB_DIM:(i + 1) * EMB_DIM, :]
        for j in range(i):
            p = ei * et_ref[j * EMB_DIM:(j + 1) * EMB_DIM, :]
            inter_ref[row, :] = jnp.sum(p, axis=0)
            row += 1

    inter = inter_ref[...]
    z = jnp.dot(tw0a_ref[...], x32, preferred_element_type=f32)
    z = z + jnp.dot(tw0b_ref[...], inter, preferred_element_type=f32)
    z = jnp.maximum(z + tb0_ref[...], 0.0)
    z = jnp.maximum(jnp.dot(tw1_ref[...], z, preferred_element_type=f32)
                    + tb1_ref[...], 0.0)
    z = jnp.maximum(jnp.dot(tw2_ref[...], z, preferred_element_type=f32)
                    + tb2_ref[...], 0.0)
    z = jnp.maximum(jnp.dot(tw3_ref[...], z, preferred_element_type=f32)
                    + tb3_ref[...], 0.0)
    out_ref[...] = jnp.dot(tw4_ref[...], z, preferred_element_type=f32) + tb4_ref[...]


def _tc_forward(xt, g, bw0t, bb0, bw1t, bb1, bw2t, bb2,
                tw0at, tw0bt, tb0, tw1t, tb1, tw2t, tb2, tw3t, tb3, tw4t, tb4):
    nb = B // _BB
    full = lambda a: pl.BlockSpec(a.shape, lambda i: (0,) * a.ndim)
    weights = (bw0t, bb0, bw1t, bb1, bw2t, bb2,
               tw0at, tw0bt, tb0, tw1t, tb1, tw2t, tb2, tw3t, tb3, tw4t, tb4)
    return pl.pallas_call(
        _tc_body,
        grid=(nb,),
        in_specs=[
            pl.BlockSpec((N_DENSE, _BB), lambda i: (0, i)),
            pl.BlockSpec((_BB, N_TABLES * EMB_DIM), lambda i: (i, 0)),
        ] + [full(w) for w in weights],
        out_specs=pl.BlockSpec((1, _BB), lambda i: (0, i)),
        out_shape=jax.ShapeDtypeStruct((1, B), jnp.float32),
        scratch_shapes=[
            pltpu.VMEM((N_FEAT * EMB_DIM, _BB), jnp.float32),
            pltpu.VMEM((N_INTERACT, _BB), jnp.float32),
        ],
    )(xt, g, *weights)


def kernel(numerical_features, cat_features, emb_tables,
           bW0, bb0, bW1, bb1, bW2, bb2,
           tW0, tb0, tW1, tb1, tW2, tb2, tW3, tb3, tW4, tb4):
    offs = jnp.arange(N_TABLES, dtype=jnp.int32) * VOCAB
    flat_idx = (cat_features.astype(jnp.int32).T + offs[None, :]).reshape(-1)
    table_flat = emb_tables.reshape(N_TABLES * VOCAB, EMB_DIM)
    gathered = _sc_gather(table_flat, flat_idx)
    g = gathered.reshape(B, N_TABLES * EMB_DIM)

    col = lambda v: v.reshape(-1, 1)
    out = _tc_forward(
        numerical_features.T, g,
        bW0.T, col(bb0), bW1.T, col(bb1), bW2.T, col(bb2),
        tW0[:EMB_DIM].T, tW0[EMB_DIM:].T, col(tb0),
        tW1.T, col(tb1), tW2.T, col(tb2), tW3.T, col(tb3), tW4.T, col(tb4),
    )
    return out.T


# speedup vs baseline: 2.0872x; 2.0872x over previous
import functools

import jax
import jax.numpy as jnp
from jax import lax
from jax.experimental import pallas as pl
from jax.experimental.pallas import tpu as pltpu
from jax.experimental.pallas import tpu_sc as plsc

B = 4096
N_DENSE = 13
N_TABLES = 26
VOCAB = 100000
EMB_DIM = 32
N_FEAT = N_TABLES + 1
N_INTERACT = N_FEAT * (N_FEAT - 1) // 2
PACK = 128 // EMB_DIM

_SC_NUM_CORES = 2
_SC_NUM_SUBCORES = 16
_NW = _SC_NUM_CORES * _SC_NUM_SUBCORES
_CHUNK = 416

_BB = 512


def _sc_gather(table128, idx_flat):
    ni = idx_flat.shape[0]
    b_per_w = ni // _NW
    n_chunks = b_per_w // _CHUNK
    mesh = plsc.VectorSubcoreMesh(core_axis_name="c", subcore_axis_name="s")

    @functools.partial(
        pl.kernel,
        mesh=mesh,
        out_type=jax.ShapeDtypeStruct((ni, 128), jnp.float32),
        scratch_types=[
            pltpu.VMEM((_CHUNK,), jnp.int32),
            pltpu.VMEM((_CHUNK, 128), jnp.float32),
            pltpu.SemaphoreType.DMA,
        ],
    )
    def gather_kernel(table_hbm, idx_hbm, out_hbm, idx_v, rows_v, sem):
        wid = lax.axis_index("s") * _SC_NUM_CORES + lax.axis_index("c")
        wbase = wid * b_per_w

        @pl.loop(0, n_chunks)
        def _(c):
            base = wbase + c * _CHUNK
            pltpu.sync_copy(idx_hbm.at[pl.ds(base, _CHUNK)], idx_v)
            pltpu.async_copy(table_hbm.at[idx_v], rows_v, sem).wait()
            pltpu.sync_copy(rows_v, out_hbm.at[pl.ds(base, _CHUNK)])

    return gather_kernel(table128, idx_flat)


def _tc_body(xt_ref, g_ref, off_ref,
             bw0_ref, bb0_ref, bw1_ref, bb1_ref, bw2_ref, bb2_ref,
             tw0a_ref, tw0b_ref, tb0_ref, tw1_ref, tb1_ref,
             tw2_ref, tb2_ref, tw3_ref, tb3_ref, tw4_ref, tb4_ref,
             out_ref, et_ref, inter_ref):
    f32 = jnp.float32
    x = xt_ref[...]
    h = jnp.maximum(jnp.dot(bw0_ref[...], x, preferred_element_type=f32)
                    + bb0_ref[...], 0.0)
    h = jnp.maximum(jnp.dot(bw1_ref[...], h, preferred_element_type=f32)
                    + bb1_ref[...], 0.0)
    x32 = jnp.maximum(jnp.dot(bw2_ref[...], h, preferred_element_type=f32)
                      + bb2_ref[...], 0.0)

    gt = jnp.transpose(g_ref[...].reshape(_BB, N_TABLES * 128))
    for t in range(N_TABLES):
        sel = jnp.zeros((EMB_DIM, _BB), dtype=f32)
        offt = off_ref[t:t + 1, :]
        for k in range(PACK):
            mk = (offt == k).astype(f32)
            sel = sel + gt[t * 128 + k * EMB_DIM:t * 128 + (k + 1) * EMB_DIM, :] * mk
        et_ref[t * EMB_DIM:(t + 1) * EMB_DIM, :] = sel
    et_ref[N_TABLES * EMB_DIM:, :] = x32

    row = 0
    for i in range(1, N_FEAT):
        ei = et_ref[i * EMB_DIM:(i + 1) * EMB_DIM, :]
        for j in range(i):
            p = ei * et_ref[j * EMB_DIM:(j + 1) * EMB_DIM, :]
            inter_ref[row, :] = jnp.sum(p, axis=0)
            row += 1

    inter = inter_ref[...]
    z = jnp.dot(tw0a_ref[...], x32, preferred_element_type=f32)
    z = z + jnp.dot(tw0b_ref[...], inter, preferred_element_type=f32)
    z = jnp.maximum(z + tb0_ref[...], 0.0)
    z = jnp.maximum(jnp.dot(tw1_ref[...], z, preferred_element_type=f32)
                    + tb1_ref[...], 0.0)
    z = jnp.maximum(jnp.dot(tw2_ref[...], z, preferred_element_type=f32)
                    + tb2_ref[...], 0.0)
    z = jnp.maximum(jnp.dot(tw3_ref[...], z, preferred_element_type=f32)
                    + tb3_ref[...], 0.0)
    out_ref[...] = jnp.dot(tw4_ref[...], z, preferred_element_type=f32) + tb4_ref[...]


def _tc_forward(xt, g, off, bw0t, bb0, bw1t, bb1, bw2t, bb2,
                tw0at, tw0bt, tb0, tw1t, tb1, tw2t, tb2, tw3t, tb3, tw4t, tb4):
    nb = B // _BB
    full = lambda a: pl.BlockSpec(a.shape, lambda i: (0,) * a.ndim)
    weights = (bw0t, bb0, bw1t, bb1, bw2t, bb2,
               tw0at, tw0bt, tb0, tw1t, tb1, tw2t, tb2, tw3t, tb3, tw4t, tb4)
    return pl.pallas_call(
        _tc_body,
        grid=(nb,),
        in_specs=[
            pl.BlockSpec((N_DENSE, _BB), lambda i: (0, i)),
            pl.BlockSpec((_BB, N_TABLES, 128), lambda i: (i, 0, 0)),
            pl.BlockSpec((N_TABLES, _BB), lambda i: (0, i)),
        ] + [full(w) for w in weights],
        out_specs=pl.BlockSpec((1, _BB), lambda i: (0, i)),
        out_shape=jax.ShapeDtypeStruct((1, B), jnp.float32),
        scratch_shapes=[
            pltpu.VMEM((N_FEAT * EMB_DIM, _BB), jnp.float32),
            pltpu.VMEM((N_INTERACT, _BB), jnp.float32),
        ],
    )(xt, g, off, *weights)


def kernel(numerical_features, cat_features, emb_tables,
           bW0, bb0, bW1, bb1, bW2, bb2,
           tW0, tb0, tW1, tb1, tW2, tb2, tW3, tb3, tW4, tb4):
    cat = cat_features.astype(jnp.int32)
    offs = jnp.arange(N_TABLES, dtype=jnp.int32) * VOCAB
    flat = cat.T + offs[None, :]
    g_idx = (flat >> 2).reshape(-1)
    sub = cat & 3
    table128 = emb_tables.reshape(N_TABLES * VOCAB // PACK, 128)
    gathered = _sc_gather(table128, g_idx)
    g = gathered.reshape(B, N_TABLES, 128)

    col = lambda v: v.reshape(-1, 1)
    out = _tc_forward(
        numerical_features.T, g, sub,
        bW0.T, col(bb0), bW1.T, col(bb1), bW2.T, col(bb2),
        tW0[:EMB_DIM].T, tW0[EMB_DIM:].T, col(tb0),
        tW1.T, col(tb1), tW2.T, col(tb2), tW3.T, col(tb3), tW4.T, col(tb4),
    )
    return out.T


# speedup vs baseline: 5.1300x; 2.4578x over previous
import functools

import jax
import jax.numpy as jnp
from jax import lax
from jax.experimental import pallas as pl
from jax.experimental.pallas import tpu as pltpu
from jax.experimental.pallas import tpu_sc as plsc

B = 4096
N_DENSE = 13
N_TABLES = 26
VOCAB = 100000
EMB_DIM = 32
N_FEAT = N_TABLES + 1
N_INTERACT = N_FEAT * (N_FEAT - 1) // 2
PACK = 4
PIECE = 24960
MAIN_ROWS = N_TABLES * PIECE
TAIL_ROWS_PER_TABLE = 64
NPACKED = MAIN_ROWS + N_TABLES * TAIL_ROWS_PER_TABLE

_SC_NUM_CORES = 2
_SC_NUM_SUBCORES = 16
_NW = _SC_NUM_CORES * _SC_NUM_SUBCORES
_CHUNK = 416

_BB = 512
_NB = B // _BB


_RSTEPS = 3
_RROWS = PIECE // _RSTEPS


def _repack_main_body(src_hbm, out_ref, xbuf, sems):
    t = pl.program_id(0)
    s = pl.program_id(1)
    cps = []
    for k in range(PACK):
        cp = pltpu.make_async_copy(
            src_hbm.at[t, :, pl.ds(k * PIECE + s * _RROWS, _RROWS)],
            xbuf.at[pl.ds(k * EMB_DIM, EMB_DIM), :],
            sems.at[k],
        )
        cp.start()
        cps.append(cp)
    for cp in cps:
        cp.wait()
    out_ref[...] = jnp.transpose(xbuf[...])


def _repack_tail_body(src_hbm, prev_ref, out_ref, buf0, buf1, sems):
    del prev_ref
    t = pl.program_id(0)
    cp0 = pltpu.make_async_copy(
        src_hbm.at[t, :, pl.ds(PACK * PIECE, 128)], buf0, sems.at[0])
    cp1 = pltpu.make_async_copy(
        src_hbm.at[t, :, pl.ds(PACK * PIECE + 128, 32)], buf1, sems.at[1])
    cp0.start()
    cp1.start()
    cp0.wait()
    cp1.wait()
    t0 = jnp.transpose(buf0[...])
    out_ref[0:32, :] = jnp.concatenate(
        [t0[k * 32:(k + 1) * 32, :] for k in range(PACK)], axis=1)
    t1 = jnp.transpose(buf1[...])
    pad = jnp.zeros((32, 128 - EMB_DIM), dtype=jnp.float32)
    out_ref[32:64, :] = jnp.concatenate([t1, pad], axis=1)


def _repack(tables_dm):
    main = pl.pallas_call(
        _repack_main_body,
        grid=(N_TABLES, _RSTEPS),
        in_specs=[pl.BlockSpec(memory_space=pl.ANY)],
        out_specs=pl.BlockSpec((_RROWS, PACK * EMB_DIM),
                               lambda t, s: (t * _RSTEPS + s, 0)),
        out_shape=jax.ShapeDtypeStruct((NPACKED, PACK * EMB_DIM), jnp.float32),
        scratch_shapes=[
            pltpu.VMEM((PACK * EMB_DIM, _RROWS), jnp.float32),
            pltpu.SemaphoreType.DMA((PACK,)),
        ],
    )(tables_dm)
    return pl.pallas_call(
        _repack_tail_body,
        grid=(N_TABLES,),
        in_specs=[pl.BlockSpec(memory_space=pl.ANY),
                  pl.BlockSpec(memory_space=pl.ANY)],
        out_specs=pl.BlockSpec((TAIL_ROWS_PER_TABLE, PACK * EMB_DIM),
                               lambda t: (MAIN_ROWS // TAIL_ROWS_PER_TABLE + t, 0)),
        out_shape=jax.ShapeDtypeStruct((NPACKED, PACK * EMB_DIM), jnp.float32),
        input_output_aliases={1: 0},
        scratch_shapes=[
            pltpu.VMEM((EMB_DIM, 128), jnp.float32),
            pltpu.VMEM((EMB_DIM, 32), jnp.float32),
            pltpu.SemaphoreType.DMA((2,)),
        ],
    )(tables_dm, main)


def _sc_gather(table128, idx_flat):
    ni = idx_flat.shape[0]
    b_per_w = ni // _NW
    n_chunks = b_per_w // _CHUNK
    mesh = plsc.VectorSubcoreMesh(core_axis_name="c", subcore_axis_name="s")

    @functools.partial(
        pl.kernel,
        mesh=mesh,
        out_type=jax.ShapeDtypeStruct((ni, 128), jnp.float32),
        scratch_types=[
            pltpu.VMEM((_CHUNK,), jnp.int32),
            pltpu.VMEM((_CHUNK, 128), jnp.float32),
            pltpu.SemaphoreType.DMA,
        ],
    )
    def gather_kernel(table_hbm, idx_hbm, out_hbm, idx_v, rows_v, sem):
        wid = lax.axis_index("s") * _SC_NUM_CORES + lax.axis_index("c")
        wbase = wid * b_per_w

        @pl.loop(0, n_chunks)
        def _(c):
            base = wbase + c * _CHUNK
            pltpu.sync_copy(idx_hbm.at[pl.ds(base, _CHUNK)], idx_v)
            pltpu.async_copy(table_hbm.at[idx_v], rows_v, sem).wait()
            pltpu.sync_copy(rows_v, out_hbm.at[pl.ds(base, _CHUNK)])

    return gather_kernel(table128, idx_flat)


def _tc_body(xt_ref, g_ref, off_ref,
             bw0_ref, bb0_ref, bw1_ref, bb1_ref, bw2_ref, bb2_ref,
             tw0a_ref, tw0b_ref, tb0_ref, tw1_ref, tb1_ref,
             tw2_ref, tb2_ref, tw3_ref, tb3_ref, tw4_ref, tb4_ref,
             out_ref, et_ref, inter_ref):
    f32 = jnp.float32
    x = xt_ref[...]
    h = jnp.maximum(jnp.dot(bw0_ref[...], x, preferred_element_type=f32)
                    + bb0_ref[...], 0.0)
    h = jnp.maximum(jnp.dot(bw1_ref[...], h, preferred_element_type=f32)
                    + bb1_ref[...], 0.0)
    x32 = jnp.maximum(jnp.dot(bw2_ref[...], h, preferred_element_type=f32)
                      + bb2_ref[...], 0.0)

    for t in range(N_TABLES):
        st = jnp.transpose(g_ref[t * _BB:(t + 1) * _BB, :])
        offt = off_ref[0, t:t + 1, :]
        sel = jnp.zeros((EMB_DIM, _BB), dtype=f32)
        for k in range(PACK):
            mk = (offt == k).astype(f32)
            sel = sel + st[k * EMB_DIM:(k + 1) * EMB_DIM, :] * mk
        et_ref[t * EMB_DIM:(t + 1) * EMB_DIM, :] = sel
    et_ref[N_TABLES * EMB_DIM:, :] = x32

    row = 0
    for i in range(1, N_FEAT):
        ei = et_ref[i * EMB_DIM:(i + 1) * EMB_DIM, :]
        for j in range(i):
            p = ei * et_ref[j * EMB_DIM:(j + 1) * EMB_DIM, :]
            inter_ref[row, :] = jnp.sum(p, axis=0)
            row += 1

    inter = inter_ref[...]
    z = jnp.dot(tw0a_ref[...], x32, preferred_element_type=f32)
    z = z + jnp.dot(tw0b_ref[...], inter, preferred_element_type=f32)
    z = jnp.maximum(z + tb0_ref[...], 0.0)
    z = jnp.maximum(jnp.dot(tw1_ref[...], z, preferred_element_type=f32)
                    + tb1_ref[...], 0.0)
    z = jnp.maximum(jnp.dot(tw2_ref[...], z, preferred_element_type=f32)
                    + tb2_ref[...], 0.0)
    z = jnp.maximum(jnp.dot(tw3_ref[...], z, preferred_element_type=f32)
                    + tb3_ref[...], 0.0)
    out_ref[...] = jnp.dot(tw4_ref[...], z, preferred_element_type=f32) + tb4_ref[...]


def _tc_forward(xt, g, off, bw0t, bb0, bw1t, bb1, bw2t, bb2,
                tw0at, tw0bt, tb0, tw1t, tb1, tw2t, tb2, tw3t, tb3, tw4t, tb4):
    full = lambda a: pl.BlockSpec(a.shape, lambda i: (0,) * a.ndim)
    weights = (bw0t, bb0, bw1t, bb1, bw2t, bb2,
               tw0at, tw0bt, tb0, tw1t, tb1, tw2t, tb2, tw3t, tb3, tw4t, tb4)
    return pl.pallas_call(
        _tc_body,
        grid=(_NB,),
        in_specs=[
            pl.BlockSpec((N_DENSE, _BB), lambda i: (0, i)),
            pl.BlockSpec((N_TABLES * _BB, 128), lambda i: (i, 0)),
            pl.BlockSpec((1, N_TABLES, _BB), lambda i: (i, 0, 0)),
        ] + [full(w) for w in weights],
        out_specs=pl.BlockSpec((1, _BB), lambda i: (0, i)),
        out_shape=jax.ShapeDtypeStruct((1, B), jnp.float32),
        scratch_shapes=[
            pltpu.VMEM((N_FEAT * EMB_DIM, _BB), jnp.float32),
            pltpu.VMEM((N_INTERACT, _BB), jnp.float32),
        ],
    )(xt, g, off, *weights)


def kernel(numerical_features, cat_features, emb_tables,
           bW0, bb0, bW1, bb1, bW2, bb2,
           tW0, tb0, tW1, tb1, tW2, tb2, tW3, tb3, tW4, tb4):
    cat = cat_features.astype(jnp.int32)
    toff = jnp.arange(N_TABLES, dtype=jnp.int32)[:, None]
    main_r = toff * PIECE + cat % PIECE
    main_k = cat // PIECE
    cp = cat - PACK * PIECE
    tail_r = jnp.where(cp >= 128,
                       MAIN_ROWS + toff * TAIL_ROWS_PER_TABLE + 32 + (cp - 128),
                       MAIN_ROWS + toff * TAIL_ROWS_PER_TABLE + (cp & 31))
    tail_k = jnp.where(cp >= 128, 0, cp >> 5)
    is_tail = cp >= 0
    rows = jnp.where(is_tail, tail_r, main_r)
    offk = jnp.where(is_tail, tail_k, main_k)
    g_idx = rows.reshape(N_TABLES, _NB, _BB).transpose(1, 0, 2).reshape(-1)
    off = offk.reshape(N_TABLES, _NB, _BB).transpose(1, 0, 2)

    tables_dm = jnp.swapaxes(emb_tables, 1, 2)
    table128 = _repack(tables_dm)
    gathered = _sc_gather(table128, g_idx)

    col = lambda v: v.reshape(-1, 1)
    out = _tc_forward(
        numerical_features.T, gathered, off,
        bW0.T, col(bb0), bW1.T, col(bb1), bW2.T, col(bb2),
        tW0[:EMB_DIM].T, tW0[EMB_DIM:].T, col(tb0),
        tW1.T, col(tb1), tW2.T, col(tb2), tW3.T, col(tb3), tW4.T, col(tb4),
    )
    return out.T


# speedup vs baseline: 7.5701x; 1.4757x over previous
import functools

import jax
import jax.numpy as jnp
from jax import lax
from jax.experimental import pallas as pl
from jax.experimental.pallas import tpu as pltpu
from jax.experimental.pallas import tpu_sc as plsc

B = 4096
N_DENSE = 13
N_TABLES = 26
VOCAB = 100000
EMB_DIM = 32
N_FEAT = N_TABLES + 1
N_INTERACT = N_FEAT * (N_FEAT - 1) // 2
PACK = 4
PIECE = 24960
MAIN_ROWS = N_TABLES * PIECE
TAIL_ROWS_PER_TABLE = 64
NPACKED = MAIN_ROWS + N_TABLES * TAIL_ROWS_PER_TABLE

_SC_NUM_CORES = 2
_SC_NUM_SUBCORES = 16
_NW = _SC_NUM_CORES * _SC_NUM_SUBCORES
_CHUNK = 416

_BB = 512
_NB = B // _BB


_RSTEPS = 3
_RROWS = PIECE // _RSTEPS


def _repack_main_body(q0_ref, q1_ref, q2_ref, q3_ref, out_ref):
    xcat = jnp.concatenate(
        [q0_ref[0], q1_ref[0], q2_ref[0], q3_ref[0]], axis=0)
    out_ref[...] = jnp.transpose(xcat)


def _repack_tail_body(src_hbm, prev_ref, out_ref, buf0, buf1, sems):
    del prev_ref
    t = pl.program_id(0)
    cp0 = pltpu.make_async_copy(
        src_hbm.at[t, :, pl.ds(PACK * PIECE, 128)], buf0, sems.at[0])
    cp1 = pltpu.make_async_copy(
        src_hbm.at[t, :, pl.ds(PACK * PIECE + 128, 32)], buf1, sems.at[1])
    cp0.start()
    cp1.start()
    cp0.wait()
    cp1.wait()
    t0 = jnp.transpose(buf0[...])
    out_ref[0:32, :] = jnp.concatenate(
        [t0[k * 32:(k + 1) * 32, :] for k in range(PACK)], axis=1)
    t1 = jnp.transpose(buf1[...])
    pad = jnp.zeros((32, 128 - EMB_DIM), dtype=jnp.float32)
    out_ref[32:64, :] = jnp.concatenate([t1, pad], axis=1)


def _repack(tables_dm):
    def qspec(k):
        return pl.BlockSpec((1, EMB_DIM, _RROWS),
                            lambda t, s, _k=k: (t, 0, _k * _RSTEPS + s))

    main = pl.pallas_call(
        _repack_main_body,
        grid=(N_TABLES, _RSTEPS),
        in_specs=[qspec(k) for k in range(PACK)],
        out_specs=pl.BlockSpec((_RROWS, PACK * EMB_DIM),
                               lambda t, s: (t * _RSTEPS + s, 0)),
        out_shape=jax.ShapeDtypeStruct((NPACKED, PACK * EMB_DIM), jnp.float32),
    )(tables_dm, tables_dm, tables_dm, tables_dm)
    return pl.pallas_call(
        _repack_tail_body,
        grid=(N_TABLES,),
        in_specs=[pl.BlockSpec(memory_space=pl.ANY),
                  pl.BlockSpec(memory_space=pl.ANY)],
        out_specs=pl.BlockSpec((TAIL_ROWS_PER_TABLE, PACK * EMB_DIM),
                               lambda t: (MAIN_ROWS // TAIL_ROWS_PER_TABLE + t, 0)),
        out_shape=jax.ShapeDtypeStruct((NPACKED, PACK * EMB_DIM), jnp.float32),
        input_output_aliases={1: 0},
        scratch_shapes=[
            pltpu.VMEM((EMB_DIM, 128), jnp.float32),
            pltpu.VMEM((EMB_DIM, 32), jnp.float32),
            pltpu.SemaphoreType.DMA((2,)),
        ],
    )(tables_dm, main)


def _sc_gather(table128, idx_flat):
    ni = idx_flat.shape[0]
    b_per_w = ni // _NW
    n_chunks = b_per_w // _CHUNK
    mesh = plsc.VectorSubcoreMesh(core_axis_name="c", subcore_axis_name="s")

    @functools.partial(
        pl.kernel,
        mesh=mesh,
        out_type=jax.ShapeDtypeStruct((ni, 128), jnp.float32),
        scratch_types=[
            pltpu.VMEM((_CHUNK,), jnp.int32),
            pltpu.VMEM((_CHUNK, 128), jnp.float32),
            pltpu.SemaphoreType.DMA,
        ],
    )
    def gather_kernel(table_hbm, idx_hbm, out_hbm, idx_v, rows_v, sem):
        wid = lax.axis_index("s") * _SC_NUM_CORES + lax.axis_index("c")
        wbase = wid * b_per_w

        @pl.loop(0, n_chunks)
        def _(c):
            base = wbase + c * _CHUNK
            pltpu.sync_copy(idx_hbm.at[pl.ds(base, _CHUNK)], idx_v)
            pltpu.async_copy(table_hbm.at[idx_v], rows_v, sem).wait()
            pltpu.sync_copy(rows_v, out_hbm.at[pl.ds(base, _CHUNK)])

    return gather_kernel(table128, idx_flat)


def _tc_body(xt_ref, g_ref, off_ref,
             bw0_ref, bb0_ref, bw1_ref, bb1_ref, bw2_ref, bb2_ref,
             tw0a_ref, tw0b_ref, tb0_ref, tw1_ref, tb1_ref,
             tw2_ref, tb2_ref, tw3_ref, tb3_ref, tw4_ref, tb4_ref,
             out_ref, et_ref, inter_ref):
    f32 = jnp.float32
    x = xt_ref[...]
    h = jnp.maximum(jnp.dot(bw0_ref[...], x, preferred_element_type=f32)
                    + bb0_ref[...], 0.0)
    h = jnp.maximum(jnp.dot(bw1_ref[...], h, preferred_element_type=f32)
                    + bb1_ref[...], 0.0)
    x32 = jnp.maximum(jnp.dot(bw2_ref[...], h, preferred_element_type=f32)
                      + bb2_ref[...], 0.0)

    for t in range(N_TABLES):
        st = jnp.transpose(g_ref[t * _BB:(t + 1) * _BB, :])
        offt = off_ref[0, t:t + 1, :]
        sel = jnp.zeros((EMB_DIM, _BB), dtype=f32)
        for k in range(PACK):
            mk = (offt == k).astype(f32)
            sel = sel + st[k * EMB_DIM:(k + 1) * EMB_DIM, :] * mk
        et_ref[t * EMB_DIM:(t + 1) * EMB_DIM, :] = sel
    et_ref[N_TABLES * EMB_DIM:, :] = x32

    row = 0
    for i in range(1, N_FEAT):
        ei = et_ref[i * EMB_DIM:(i + 1) * EMB_DIM, :]
        for j in range(i):
            p = ei * et_ref[j * EMB_DIM:(j + 1) * EMB_DIM, :]
            inter_ref[row, :] = jnp.sum(p, axis=0)
            row += 1

    inter = inter_ref[...]
    z = jnp.dot(tw0a_ref[...], x32, preferred_element_type=f32)
    z = z + jnp.dot(tw0b_ref[...], inter, preferred_element_type=f32)
    z = jnp.maximum(z + tb0_ref[...], 0.0)
    z = jnp.maximum(jnp.dot(tw1_ref[...], z, preferred_element_type=f32)
                    + tb1_ref[...], 0.0)
    z = jnp.maximum(jnp.dot(tw2_ref[...], z, preferred_element_type=f32)
                    + tb2_ref[...], 0.0)
    z = jnp.maximum(jnp.dot(tw3_ref[...], z, preferred_element_type=f32)
                    + tb3_ref[...], 0.0)
    out_ref[...] = jnp.dot(tw4_ref[...], z, preferred_element_type=f32) + tb4_ref[...]


def _tc_forward(xt, g, off, bw0t, bb0, bw1t, bb1, bw2t, bb2,
                tw0at, tw0bt, tb0, tw1t, tb1, tw2t, tb2, tw3t, tb3, tw4t, tb4):
    full = lambda a: pl.BlockSpec(a.shape, lambda i: (0,) * a.ndim)
    weights = (bw0t, bb0, bw1t, bb1, bw2t, bb2,
               tw0at, tw0bt, tb0, tw1t, tb1, tw2t, tb2, tw3t, tb3, tw4t, tb4)
    return pl.pallas_call(
        _tc_body,
        grid=(_NB,),
        in_specs=[
            pl.BlockSpec((N_DENSE, _BB), lambda i: (0, i)),
            pl.BlockSpec((N_TABLES * _BB, 128), lambda i: (i, 0)),
            pl.BlockSpec((1, N_TABLES, _BB), lambda i: (i, 0, 0)),
        ] + [full(w) for w in weights],
        out_specs=pl.BlockSpec((1, _BB), lambda i: (0, i)),
        out_shape=jax.ShapeDtypeStruct((1, B), jnp.float32),
        scratch_shapes=[
            pltpu.VMEM((N_FEAT * EMB_DIM, _BB), jnp.float32),
            pltpu.VMEM((N_INTERACT, _BB), jnp.float32),
        ],
    )(xt, g, off, *weights)


def kernel(numerical_features, cat_features, emb_tables,
           bW0, bb0, bW1, bb1, bW2, bb2,
           tW0, tb0, tW1, tb1, tW2, tb2, tW3, tb3, tW4, tb4):
    cat = cat_features.astype(jnp.int32)
    toff = jnp.arange(N_TABLES, dtype=jnp.int32)[:, None]
    main_r = toff * PIECE + cat % PIECE
    main_k = cat // PIECE
    cp = cat - PACK * PIECE
    tail_r = jnp.where(cp >= 128,
                       MAIN_ROWS + toff * TAIL_ROWS_PER_TABLE + 32 + (cp - 128),
                       MAIN_ROWS + toff * TAIL_ROWS_PER_TABLE + (cp & 31))
    tail_k = jnp.where(cp >= 128, 0, cp >> 5)
    is_tail = cp >= 0
    rows = jnp.where(is_tail, tail_r, main_r)
    offk = jnp.where(is_tail, tail_k, main_k)
    g_idx = rows.reshape(N_TABLES, _NB, _BB).transpose(1, 0, 2).reshape(-1)
    off = offk.reshape(N_TABLES, _NB, _BB).transpose(1, 0, 2)

    tables_dm = jnp.swapaxes(emb_tables, 1, 2)
    table128 = _repack(tables_dm)
    gathered = _sc_gather(table128, g_idx)

    col = lambda v: v.reshape(-1, 1)
    out = _tc_forward(
        numerical_features.T, gathered, off,
        bW0.T, col(bb0), bW1.T, col(bb1), bW2.T, col(bb2),
        tW0[:EMB_DIM].T, tW0[EMB_DIM:].T, col(tb0),
        tW1.T, col(tb1), tW2.T, col(tb2), tW3.T, col(tb3), tW4.T, col(tb4),
    )
    return out.T


# speedup vs baseline: 7.5740x; 1.0005x over previous
import functools

import jax
import jax.numpy as jnp
from jax import lax
from jax.experimental import pallas as pl
from jax.experimental.pallas import tpu as pltpu
from jax.experimental.pallas import tpu_sc as plsc

B = 4096
N_DENSE = 13
N_TABLES = 26
VOCAB = 100000
EMB_DIM = 32
N_FEAT = N_TABLES + 1
N_INTERACT = N_FEAT * (N_FEAT - 1) // 2
PACK = 4
PIECE = 24960
MAIN_ROWS = N_TABLES * PIECE
TAIL_ROWS_PER_TABLE = 64
NPACKED = MAIN_ROWS + N_TABLES * TAIL_ROWS_PER_TABLE

_SC_NUM_CORES = 2
_SC_NUM_SUBCORES = 16
_NW = _SC_NUM_CORES * _SC_NUM_SUBCORES
_CHUNK = 416

_BB = 512
_NB = B // _BB


_RSTEPS = 3
_RROWS = PIECE // _RSTEPS


def _repack_main_body(q0_ref, q1_ref, q2_ref, q3_ref, out_ref):
    xcat = jnp.concatenate(
        [q0_ref[0], q1_ref[0], q2_ref[0], q3_ref[0]], axis=0)
    out_ref[...] = jnp.transpose(xcat)


def _repack_tail_body(src_hbm, prev_ref, out_ref, buf0, buf1, sems):
    del prev_ref
    t = pl.program_id(0)
    cp0 = pltpu.make_async_copy(
        src_hbm.at[t, :, pl.ds(PACK * PIECE, 128)], buf0, sems.at[0])
    cp1 = pltpu.make_async_copy(
        src_hbm.at[t, :, pl.ds(PACK * PIECE + 128, 32)], buf1, sems.at[1])
    cp0.start()
    cp1.start()
    cp0.wait()
    cp1.wait()
    t0 = jnp.transpose(buf0[...])
    out_ref[0:32, :] = jnp.concatenate(
        [t0[k * 32:(k + 1) * 32, :] for k in range(PACK)], axis=1)
    t1 = jnp.transpose(buf1[...])
    pad = jnp.zeros((32, 128 - EMB_DIM), dtype=jnp.float32)
    out_ref[32:64, :] = jnp.concatenate([t1, pad], axis=1)


def _repack(tables_dm):
    def qspec(k):
        return pl.BlockSpec((1, EMB_DIM, _RROWS),
                            lambda t, s, _k=k: (t, 0, _k * _RSTEPS + s))

    main = pl.pallas_call(
        _repack_main_body,
        grid=(N_TABLES, _RSTEPS),
        in_specs=[qspec(k) for k in range(PACK)],
        out_specs=pl.BlockSpec((_RROWS, PACK * EMB_DIM),
                               lambda t, s: (t * _RSTEPS + s, 0)),
        out_shape=jax.ShapeDtypeStruct((NPACKED, PACK * EMB_DIM), jnp.float32),
        compiler_params=pltpu.CompilerParams(
            dimension_semantics=("parallel", "parallel")),
    )(tables_dm, tables_dm, tables_dm, tables_dm)
    return pl.pallas_call(
        _repack_tail_body,
        grid=(N_TABLES,),
        in_specs=[pl.BlockSpec(memory_space=pl.ANY),
                  pl.BlockSpec(memory_space=pl.ANY)],
        out_specs=pl.BlockSpec((TAIL_ROWS_PER_TABLE, PACK * EMB_DIM),
                               lambda t: (MAIN_ROWS // TAIL_ROWS_PER_TABLE + t, 0)),
        out_shape=jax.ShapeDtypeStruct((NPACKED, PACK * EMB_DIM), jnp.float32),
        input_output_aliases={1: 0},
        scratch_shapes=[
            pltpu.VMEM((EMB_DIM, 128), jnp.float32),
            pltpu.VMEM((EMB_DIM, 32), jnp.float32),
            pltpu.SemaphoreType.DMA((2,)),
        ],
    )(tables_dm, main)


def _sc_gather(table128, idx_flat):
    ni = idx_flat.shape[0]
    b_per_w = ni // _NW
    n_chunks = b_per_w // _CHUNK
    mesh = plsc.VectorSubcoreMesh(core_axis_name="c", subcore_axis_name="s")

    @functools.partial(
        pl.kernel,
        mesh=mesh,
        out_type=jax.ShapeDtypeStruct((ni, 128), jnp.float32),
        scratch_types=[
            pltpu.VMEM((_CHUNK,), jnp.int32),
            pltpu.VMEM((_CHUNK, 128), jnp.float32),
            pltpu.SemaphoreType.DMA,
        ],
    )
    def gather_kernel(table_hbm, idx_hbm, out_hbm, idx_v, rows_v, sem):
        wid = lax.axis_index("s") * _SC_NUM_CORES + lax.axis_index("c")
        wbase = wid * b_per_w

        @pl.loop(0, n_chunks)
        def _(c):
            base = wbase + c * _CHUNK
            pltpu.sync_copy(idx_hbm.at[pl.ds(base, _CHUNK)], idx_v)
            pltpu.async_copy(table_hbm.at[idx_v], rows_v, sem).wait()
            pltpu.sync_copy(rows_v, out_hbm.at[pl.ds(base, _CHUNK)])

    return gather_kernel(table128, idx_flat)


def _tc_body(xt_ref, g_ref, off_ref,
             bw0_ref, bb0_ref, bw1_ref, bb1_ref, bw2_ref, bb2_ref,
             tw0a_ref, tw0b_ref, tb0_ref, tw1_ref, tb1_ref,
             tw2_ref, tb2_ref, tw3_ref, tb3_ref, tw4_ref, tb4_ref,
             out_ref, et_ref, inter_ref):
    f32 = jnp.float32
    x = xt_ref[...]
    h = jnp.maximum(jnp.dot(bw0_ref[...], x, preferred_element_type=f32)
                    + bb0_ref[...], 0.0)
    h = jnp.maximum(jnp.dot(bw1_ref[...], h, preferred_element_type=f32)
                    + bb1_ref[...], 0.0)
    x32 = jnp.maximum(jnp.dot(bw2_ref[...], h, preferred_element_type=f32)
                      + bb2_ref[...], 0.0)

    for t in range(N_TABLES):
        st = jnp.transpose(g_ref[t * _BB:(t + 1) * _BB, :])
        offt = off_ref[0, t:t + 1, :]
        sel = jnp.zeros((EMB_DIM, _BB), dtype=f32)
        for k in range(PACK):
            mk = (offt == k).astype(f32)
            sel = sel + st[k * EMB_DIM:(k + 1) * EMB_DIM, :] * mk
        et_ref[t * EMB_DIM:(t + 1) * EMB_DIM, :] = sel
    et_ref[N_TABLES * EMB_DIM:, :] = x32

    row = 0
    for i in range(1, N_FEAT):
        ei = et_ref[i * EMB_DIM:(i + 1) * EMB_DIM, :]
        for j in range(i):
            p = ei * et_ref[j * EMB_DIM:(j + 1) * EMB_DIM, :]
            inter_ref[row, :] = jnp.sum(p, axis=0)
            row += 1

    inter = inter_ref[...]
    z = jnp.dot(tw0a_ref[...], x32, preferred_element_type=f32)
    z = z + jnp.dot(tw0b_ref[...], inter, preferred_element_type=f32)
    z = jnp.maximum(z + tb0_ref[...], 0.0)
    z = jnp.maximum(jnp.dot(tw1_ref[...], z, preferred_element_type=f32)
                    + tb1_ref[...], 0.0)
    z = jnp.maximum(jnp.dot(tw2_ref[...], z, preferred_element_type=f32)
                    + tb2_ref[...], 0.0)
    z = jnp.maximum(jnp.dot(tw3_ref[...], z, preferred_element_type=f32)
                    + tb3_ref[...], 0.0)
    out_ref[...] = jnp.dot(tw4_ref[...], z, preferred_element_type=f32) + tb4_ref[...]


def _tc_forward(xt, g, off, bw0t, bb0, bw1t, bb1, bw2t, bb2,
                tw0at, tw0bt, tb0, tw1t, tb1, tw2t, tb2, tw3t, tb3, tw4t, tb4):
    full = lambda a: pl.BlockSpec(a.shape, lambda i: (0,) * a.ndim)
    weights = (bw0t, bb0, bw1t, bb1, bw2t, bb2,
               tw0at, tw0bt, tb0, tw1t, tb1, tw2t, tb2, tw3t, tb3, tw4t, tb4)
    return pl.pallas_call(
        _tc_body,
        grid=(_NB,),
        in_specs=[
            pl.BlockSpec((N_DENSE, _BB), lambda i: (0, i)),
            pl.BlockSpec((N_TABLES * _BB, 128), lambda i: (i, 0)),
            pl.BlockSpec((1, N_TABLES, _BB), lambda i: (i, 0, 0)),
        ] + [full(w) for w in weights],
        out_specs=pl.BlockSpec((1, _BB), lambda i: (0, i)),
        out_shape=jax.ShapeDtypeStruct((1, B), jnp.float32),
        scratch_shapes=[
            pltpu.VMEM((N_FEAT * EMB_DIM, _BB), jnp.float32),
            pltpu.VMEM((N_INTERACT, _BB), jnp.float32),
        ],
        compiler_params=pltpu.CompilerParams(
            dimension_semantics=("parallel",)),
    )(xt, g, off, *weights)


def kernel(numerical_features, cat_features, emb_tables,
           bW0, bb0, bW1, bb1, bW2, bb2,
           tW0, tb0, tW1, tb1, tW2, tb2, tW3, tb3, tW4, tb4):
    cat = cat_features.astype(jnp.int32)
    toff = jnp.arange(N_TABLES, dtype=jnp.int32)[:, None]
    main_r = toff * PIECE + cat % PIECE
    main_k = cat // PIECE
    cp = cat - PACK * PIECE
    tail_r = jnp.where(cp >= 128,
                       MAIN_ROWS + toff * TAIL_ROWS_PER_TABLE + 32 + (cp - 128),
                       MAIN_ROWS + toff * TAIL_ROWS_PER_TABLE + (cp & 31))
    tail_k = jnp.where(cp >= 128, 0, cp >> 5)
    is_tail = cp >= 0
    rows = jnp.where(is_tail, tail_r, main_r)
    offk = jnp.where(is_tail, tail_k, main_k)
    g_idx = rows.reshape(N_TABLES, _NB, _BB).transpose(1, 0, 2).reshape(-1)
    off = offk.reshape(N_TABLES, _NB, _BB).transpose(1, 0, 2)

    tables_dm = jnp.swapaxes(emb_tables, 1, 2)
    table128 = _repack(tables_dm)
    gathered = _sc_gather(table128, g_idx)

    col = lambda v: v.reshape(-1, 1)
    out = _tc_forward(
        numerical_features.T, gathered, off,
        bW0.T, col(bb0), bW1.T, col(bb1), bW2.T, col(bb2),
        tW0[:EMB_DIM].T, tW0[EMB_DIM:].T, col(tb0),
        tW1.T, col(tb1), tW2.T, col(tb2), tW3.T, col(tb3), tW4.T, col(tb4),
    )
    return out.T


# speedup vs baseline: 8.2274x; 1.0863x over previous
import functools

import jax
import jax.numpy as jnp
from jax import lax
from jax.experimental import pallas as pl
from jax.experimental.pallas import tpu as pltpu
from jax.experimental.pallas import tpu_sc as plsc

B = 4096
N_DENSE = 13
N_TABLES = 26
VOCAB = 100000
EMB_DIM = 32
N_FEAT = N_TABLES + 1
N_INTERACT = N_FEAT * (N_FEAT - 1) // 2
PACK = 4
PIECE = 24960
MAIN_ROWS = N_TABLES * PIECE
TAIL_ROWS_PER_TABLE = 64
NPACKED = MAIN_ROWS + N_TABLES * TAIL_ROWS_PER_TABLE
I32_MAIN_ROWS = MAIN_ROWS // 2
I32_TAIL_PER_TABLE = TAIL_ROWS_PER_TABLE // 2
NPACKED_I32 = I32_MAIN_ROWS + N_TABLES * I32_TAIL_PER_TABLE

_SC_NUM_CORES = 2
_SC_NUM_SUBCORES = 16
_NW = _SC_NUM_CORES * _SC_NUM_SUBCORES
_CHUNK = 416

_BB = 512
_NB = B // _BB


_RSTEPS = 3
_RROWS = PIECE // _RSTEPS
_HROWS = _RROWS // 2


def _pack_bf16_pair(lo_f32_bits, hi_f32_bits):
    lo = jnp.bitwise_and(jnp.right_shift(lo_f32_bits + 32768, 16),
                         jnp.int32(0xFFFF))
    hi = jnp.bitwise_and(hi_f32_bits + 32768, jnp.int32(-65536))
    return jnp.bitwise_or(hi, lo)


def _repack_main_body(q0_ref, q1_ref, q2_ref, q3_ref, out_ref):
    xcat = jnp.concatenate(
        [q0_ref[0], q1_ref[0], q2_ref[0], q3_ref[0]], axis=0)
    y = jax.lax.bitcast_convert_type(jnp.transpose(xcat), jnp.int32)
    out_ref[...] = _pack_bf16_pair(y[:_HROWS, :], y[_HROWS:, :])


def _repack_tail_body(src_hbm, prev_ref, out_ref, buf0, buf1, sems):
    del prev_ref
    t = pl.program_id(0)
    cp0 = pltpu.make_async_copy(
        src_hbm.at[t, :, pl.ds(PACK * PIECE, 128)], buf0, sems.at[0])
    cp1 = pltpu.make_async_copy(
        src_hbm.at[t, :, pl.ds(PACK * PIECE + 128, 32)], buf1, sems.at[1])
    cp0.start()
    cp1.start()
    cp0.wait()
    cp1.wait()
    t0 = jnp.transpose(buf0[...])
    w0 = jnp.concatenate(
        [t0[k * 32:(k + 1) * 32, :] for k in range(PACK)], axis=1)
    w0 = jax.lax.bitcast_convert_type(w0, jnp.int32)
    out_ref[0:16, :] = _pack_bf16_pair(w0[0:16, :], w0[16:32, :])
    t1 = jnp.transpose(buf1[...])
    pad = jnp.zeros((32, 128 - EMB_DIM), dtype=jnp.float32)
    w1 = jax.lax.bitcast_convert_type(
        jnp.concatenate([t1, pad], axis=1), jnp.int32)
    out_ref[16:32, :] = _pack_bf16_pair(w1[0:16, :], w1[16:32, :])


def _repack(tables_dm):
    def qspec(k):
        return pl.BlockSpec((1, EMB_DIM, _RROWS),
                            lambda t, s, _k=k: (t, 0, _k * _RSTEPS + s))

    main = pl.pallas_call(
        _repack_main_body,
        grid=(N_TABLES, _RSTEPS),
        in_specs=[qspec(k) for k in range(PACK)],
        out_specs=pl.BlockSpec((_HROWS, PACK * EMB_DIM),
                               lambda t, s: (t * _RSTEPS + s, 0)),
        out_shape=jax.ShapeDtypeStruct((NPACKED_I32, PACK * EMB_DIM), jnp.int32),
        compiler_params=pltpu.CompilerParams(
            dimension_semantics=("parallel", "parallel")),
    )(tables_dm, tables_dm, tables_dm, tables_dm)
    return pl.pallas_call(
        _repack_tail_body,
        grid=(N_TABLES,),
        in_specs=[pl.BlockSpec(memory_space=pl.ANY),
                  pl.BlockSpec(memory_space=pl.ANY)],
        out_specs=pl.BlockSpec((I32_TAIL_PER_TABLE, PACK * EMB_DIM),
                               lambda t: (I32_MAIN_ROWS // I32_TAIL_PER_TABLE + t, 0)),
        out_shape=jax.ShapeDtypeStruct((NPACKED_I32, PACK * EMB_DIM), jnp.int32),
        input_output_aliases={1: 0},
        scratch_shapes=[
            pltpu.VMEM((EMB_DIM, 128), jnp.float32),
            pltpu.VMEM((EMB_DIM, 32), jnp.float32),
            pltpu.SemaphoreType.DMA((2,)),
        ],
    )(tables_dm, main)


def _sc_gather(table128, idx_flat):
    ni = idx_flat.shape[0]
    b_per_w = ni // _NW
    n_chunks = b_per_w // _CHUNK
    mesh = plsc.VectorSubcoreMesh(core_axis_name="c", subcore_axis_name="s")

    @functools.partial(
        pl.kernel,
        mesh=mesh,
        out_type=jax.ShapeDtypeStruct((ni, 128), jnp.int32),
        scratch_types=[
            pltpu.VMEM((_CHUNK,), jnp.int32),
            pltpu.VMEM((_CHUNK, 128), jnp.int32),
            pltpu.SemaphoreType.DMA,
        ],
    )
    def gather_kernel(table_hbm, idx_hbm, out_hbm, idx_v, rows_v, sem):
        wid = lax.axis_index("s") * _SC_NUM_CORES + lax.axis_index("c")
        wbase = wid * b_per_w

        @pl.loop(0, n_chunks)
        def _(c):
            base = wbase + c * _CHUNK
            pltpu.sync_copy(idx_hbm.at[pl.ds(base, _CHUNK)], idx_v)
            pltpu.async_copy(table_hbm.at[idx_v], rows_v, sem).wait()
            pltpu.sync_copy(rows_v, out_hbm.at[pl.ds(base, _CHUNK)])

    return gather_kernel(table128, idx_flat)


def _tc_body(xt_ref, g_ref, off_ref,
             bw0_ref, bb0_ref, bw1_ref, bb1_ref, bw2_ref, bb2_ref,
             tw0a_ref, tw0b_ref, tb0_ref, tw1_ref, tb1_ref,
             tw2_ref, tb2_ref, tw3_ref, tb3_ref, tw4_ref, tb4_ref,
             out_ref, et_ref, inter_ref):
    f32 = jnp.float32
    x = xt_ref[...]
    h = jnp.maximum(jnp.dot(bw0_ref[...], x, preferred_element_type=f32)
                    + bb0_ref[...], 0.0)
    h = jnp.maximum(jnp.dot(bw1_ref[...], h, preferred_element_type=f32)
                    + bb1_ref[...], 0.0)
    x32 = jnp.maximum(jnp.dot(bw2_ref[...], h, preferred_element_type=f32)
                      + bb2_ref[...], 0.0)

    for t in range(N_TABLES):
        sti = jnp.transpose(g_ref[t * _BB:(t + 1) * _BB, :])
        lowf = jax.lax.bitcast_convert_type(
            jnp.left_shift(sti, 16), f32)
        highf = jax.lax.bitcast_convert_type(
            jnp.bitwise_and(sti, jnp.int32(-65536)), f32)
        offt = off_ref[0, t:t + 1, :]
        sel = jnp.zeros((EMB_DIM, _BB), dtype=f32)
        for j in range(2 * PACK):
            k, half = j >> 1, j & 1
            srcf = highf if half else lowf
            mk = (offt == j).astype(f32)
            sel = sel + srcf[k * EMB_DIM:(k + 1) * EMB_DIM, :] * mk
        et_ref[t * EMB_DIM:(t + 1) * EMB_DIM, :] = sel
    et_ref[N_TABLES * EMB_DIM:, :] = x32

    row = 0
    for i in range(1, N_FEAT):
        ei = et_ref[i * EMB_DIM:(i + 1) * EMB_DIM, :]
        for j in range(i):
            p = ei * et_ref[j * EMB_DIM:(j + 1) * EMB_DIM, :]
            inter_ref[row, :] = jnp.sum(p, axis=0)
            row += 1

    inter = inter_ref[...]
    z = jnp.dot(tw0a_ref[...], x32, preferred_element_type=f32)
    z = z + jnp.dot(tw0b_ref[...], inter, preferred_element_type=f32)
    z = jnp.maximum(z + tb0_ref[...], 0.0)
    z = jnp.maximum(jnp.dot(tw1_ref[...], z, preferred_element_type=f32)
                    + tb1_ref[...], 0.0)
    z = jnp.maximum(jnp.dot(tw2_ref[...], z, preferred_element_type=f32)
                    + tb2_ref[...], 0.0)
    z = jnp.maximum(jnp.dot(tw3_ref[...], z, preferred_element_type=f32)
                    + tb3_ref[...], 0.0)
    out_ref[...] = jnp.dot(tw4_ref[...], z, preferred_element_type=f32) + tb4_ref[...]


def _tc_forward(xt, g, off, bw0t, bb0, bw1t, bb1, bw2t, bb2,
                tw0at, tw0bt, tb0, tw1t, tb1, tw2t, tb2, tw3t, tb3, tw4t, tb4):
    full = lambda a: pl.BlockSpec(a.shape, lambda i: (0,) * a.ndim)
    weights = (bw0t, bb0, bw1t, bb1, bw2t, bb2,
               tw0at, tw0bt, tb0, tw1t, tb1, tw2t, tb2, tw3t, tb3, tw4t, tb4)
    return pl.pallas_call(
        _tc_body,
        grid=(_NB,),
        in_specs=[
            pl.BlockSpec((N_DENSE, _BB), lambda i: (0, i)),
            pl.BlockSpec((N_TABLES * _BB, 128), lambda i: (i, 0)),
            pl.BlockSpec((1, N_TABLES, _BB), lambda i: (i, 0, 0)),
        ] + [full(w) for w in weights],
        out_specs=pl.BlockSpec((1, _BB), lambda i: (0, i)),
        out_shape=jax.ShapeDtypeStruct((1, B), jnp.float32),
        scratch_shapes=[
            pltpu.VMEM((N_FEAT * EMB_DIM, _BB), jnp.float32),
            pltpu.VMEM((N_INTERACT, _BB), jnp.float32),
        ],
        compiler_params=pltpu.CompilerParams(
            dimension_semantics=("parallel",)),
    )(xt, g, off, *weights)


def kernel(numerical_features, cat_features, emb_tables,
           bW0, bb0, bW1, bb1, bW2, bb2,
           tW0, tb0, tW1, tb1, tW2, tb2, tW3, tb3, tW4, tb4):
    cat = cat_features.astype(jnp.int32)
    toff = jnp.arange(N_TABLES, dtype=jnp.int32)[:, None]
    q = cat % PIECE
    s_ = q // _RROWS
    r_ = q % _RROWS
    main_r = (toff * _RSTEPS + s_) * _HROWS + r_ % _HROWS
    main_k = (cat // PIECE) * 2 + r_ // _HROWS
    cp = cat - PACK * PIECE
    rt = jnp.where(cp >= 128, 32 + (cp - 128), cp & 31)
    kt = jnp.where(cp >= 128, 0, cp >> 5)
    tail_base = I32_MAIN_ROWS + toff * I32_TAIL_PER_TABLE
    tail_r = jnp.where(rt >= 32,
                       tail_base + 16 + (rt - 32) % 16,
                       tail_base + rt % 16)
    tail_half = jnp.where(rt >= 32, (rt - 32) // 16, rt // 16)
    tail_k = kt * 2 + tail_half
    is_tail = cp >= 0
    rows = jnp.where(is_tail, tail_r, main_r)
    offk = jnp.where(is_tail, tail_k, main_k)
    g_idx = rows.reshape(N_TABLES, _NB, _BB).transpose(1, 0, 2).reshape(-1)
    off = offk.reshape(N_TABLES, _NB, _BB).transpose(1, 0, 2)

    tables_dm = jnp.swapaxes(emb_tables, 1, 2)
    table128 = _repack(tables_dm)
    gathered = _sc_gather(table128, g_idx)

    col = lambda v: v.reshape(-1, 1)
    out = _tc_forward(
        numerical_features.T, gathered, off,
        bW0.T, col(bb0), bW1.T, col(bb1), bW2.T, col(bb2),
        tW0[:EMB_DIM].T, tW0[EMB_DIM:].T, col(tb0),
        tW1.T, col(tb1), tW2.T, col(tb2), tW3.T, col(tb3), tW4.T, col(tb4),
    )
    return out.T


# speedup vs baseline: 8.8214x; 1.0722x over previous
import functools

import jax
import jax.numpy as jnp
from jax import lax
from jax.experimental import pallas as pl
from jax.experimental.pallas import tpu as pltpu
from jax.experimental.pallas import tpu_sc as plsc

B = 4096
N_DENSE = 13
N_TABLES = 26
VOCAB = 100000
EMB_DIM = 32
N_FEAT = N_TABLES + 1
N_INTERACT = N_FEAT * (N_FEAT - 1) // 2
PACK = 4
PIECE = 24960
MAIN_ROWS = N_TABLES * PIECE
TAIL_ROWS_PER_TABLE = 64
NPACKED = MAIN_ROWS + N_TABLES * TAIL_ROWS_PER_TABLE
I32_MAIN_ROWS = MAIN_ROWS // 2
I32_TAIL_PER_TABLE = TAIL_ROWS_PER_TABLE // 2
NPACKED_I32 = I32_MAIN_ROWS + N_TABLES * I32_TAIL_PER_TABLE

_SC_NUM_CORES = 2
_SC_NUM_SUBCORES = 16
_NW = _SC_NUM_CORES * _SC_NUM_SUBCORES
_CHUNK = 416

_BB = 512
_NB = B // _BB


_RSTEPS = 3
_RROWS = PIECE // _RSTEPS
_HROWS = _RROWS // 2


def _pack_bf16_pair(lo_f32_bits, hi_f32_bits):
    lo = jnp.bitwise_and(jnp.right_shift(lo_f32_bits + 32768, 16),
                         jnp.int32(0xFFFF))
    hi = jnp.bitwise_and(hi_f32_bits + 32768, jnp.int32(-65536))
    return jnp.bitwise_or(hi, lo)


def _repack_main_body(q0_ref, q1_ref, q2_ref, q3_ref, out_ref):
    xcat = jnp.concatenate(
        [q0_ref[0], q1_ref[0], q2_ref[0], q3_ref[0]], axis=0)
    y = jax.lax.bitcast_convert_type(jnp.transpose(xcat), jnp.int32)
    out_ref[...] = _pack_bf16_pair(y[:_HROWS, :], y[_HROWS:, :])


def _repack_tail_body(src_hbm, prev_ref, out_ref, buf0, buf1, sems):
    del prev_ref
    cps = []
    for t in range(N_TABLES):
        cp0 = pltpu.make_async_copy(
            src_hbm.at[t, :, pl.ds(PACK * PIECE, 128)], buf0.at[t], sems.at[t, 0])
        cp1 = pltpu.make_async_copy(
            src_hbm.at[t, :, pl.ds(PACK * PIECE + 128, 32)], buf1.at[t], sems.at[t, 1])
        cp0.start()
        cp1.start()
        cps.extend((cp0, cp1))
    for cp in cps:
        cp.wait()
    pad = jnp.zeros((32, 128 - EMB_DIM), dtype=jnp.float32)
    for t in range(N_TABLES):
        t0 = jnp.transpose(buf0[t])
        w0 = jnp.concatenate(
            [t0[k * 32:(k + 1) * 32, :] for k in range(PACK)], axis=1)
        w0 = jax.lax.bitcast_convert_type(w0, jnp.int32)
        out_ref[32 * t:32 * t + 16, :] = _pack_bf16_pair(w0[0:16, :], w0[16:32, :])
        t1 = jnp.transpose(buf1[t])
        w1 = jax.lax.bitcast_convert_type(
            jnp.concatenate([t1, pad], axis=1), jnp.int32)
        out_ref[32 * t + 16:32 * t + 32, :] = _pack_bf16_pair(w1[0:16, :], w1[16:32, :])


def _repack(tables_dm):
    def qspec(k):
        return pl.BlockSpec((1, EMB_DIM, _RROWS),
                            lambda t, s, _k=k: (t, 0, _k * _RSTEPS + s))

    main = pl.pallas_call(
        _repack_main_body,
        grid=(N_TABLES, _RSTEPS),
        in_specs=[qspec(k) for k in range(PACK)],
        out_specs=pl.BlockSpec((_HROWS, PACK * EMB_DIM),
                               lambda t, s: (t * _RSTEPS + s, 0)),
        out_shape=jax.ShapeDtypeStruct((NPACKED_I32, PACK * EMB_DIM), jnp.int32),
        compiler_params=pltpu.CompilerParams(
            dimension_semantics=("parallel", "parallel")),
    )(tables_dm, tables_dm, tables_dm, tables_dm)
    return pl.pallas_call(
        _repack_tail_body,
        grid=(1,),
        in_specs=[pl.BlockSpec(memory_space=pl.ANY),
                  pl.BlockSpec(memory_space=pl.ANY)],
        out_specs=pl.BlockSpec((N_TABLES * I32_TAIL_PER_TABLE, PACK * EMB_DIM),
                               lambda i: (I32_MAIN_ROWS //
                                          (N_TABLES * I32_TAIL_PER_TABLE), 0)),
        out_shape=jax.ShapeDtypeStruct((NPACKED_I32, PACK * EMB_DIM), jnp.int32),
        input_output_aliases={1: 0},
        scratch_shapes=[
            pltpu.VMEM((N_TABLES, EMB_DIM, 128), jnp.float32),
            pltpu.VMEM((N_TABLES, EMB_DIM, 32), jnp.float32),
            pltpu.SemaphoreType.DMA((N_TABLES, 2)),
        ],
    )(tables_dm, main)


def _sc_gather(table128, idx_flat):
    ni = idx_flat.shape[0]
    b_per_w = ni // _NW
    n_chunks = b_per_w // _CHUNK
    mesh = plsc.VectorSubcoreMesh(core_axis_name="c", subcore_axis_name="s")

    @functools.partial(
        pl.kernel,
        mesh=mesh,
        out_type=jax.ShapeDtypeStruct((ni, 128), jnp.int32),
        scratch_types=[
            pltpu.VMEM((_CHUNK,), jnp.int32),
            pltpu.VMEM((_CHUNK, 128), jnp.int32),
            pltpu.SemaphoreType.DMA,
        ],
    )
    def gather_kernel(table_hbm, idx_hbm, out_hbm, idx_v, rows_v, sem):
        wid = lax.axis_index("s") * _SC_NUM_CORES + lax.axis_index("c")
        wbase = wid * b_per_w

        @pl.loop(0, n_chunks)
        def _(c):
            base = wbase + c * _CHUNK
            pltpu.sync_copy(idx_hbm.at[pl.ds(base, _CHUNK)], idx_v)
            pltpu.async_copy(table_hbm.at[idx_v], rows_v, sem).wait()
            pltpu.sync_copy(rows_v, out_hbm.at[pl.ds(base, _CHUNK)])

    return gather_kernel(table128, idx_flat)


def _tc_body(xt_ref, g_ref, off_ref,
             bw0_ref, bb0_ref, bw1_ref, bb1_ref, bw2_ref, bb2_ref,
             tw0a_ref, tw0b_ref, tb0_ref, tw1_ref, tb1_ref,
             tw2_ref, tb2_ref, tw3_ref, tb3_ref, tw4_ref, tb4_ref,
             out_ref, et_ref, inter_ref):
    f32 = jnp.float32
    x = xt_ref[...]
    h = jnp.maximum(jnp.dot(bw0_ref[...], x, preferred_element_type=f32)
                    + bb0_ref[...], 0.0)
    h = jnp.maximum(jnp.dot(bw1_ref[...], h, preferred_element_type=f32)
                    + bb1_ref[...], 0.0)
    x32 = jnp.maximum(jnp.dot(bw2_ref[...], h, preferred_element_type=f32)
                      + bb2_ref[...], 0.0)

    for t in range(N_TABLES):
        sti = jnp.transpose(g_ref[t * _BB:(t + 1) * _BB, :])
        lowf = jax.lax.bitcast_convert_type(
            jnp.left_shift(sti, 16), f32)
        highf = jax.lax.bitcast_convert_type(
            jnp.bitwise_and(sti, jnp.int32(-65536)), f32)
        offt = off_ref[0, t:t + 1, :]
        sel = jnp.zeros((EMB_DIM, _BB), dtype=f32)
        for j in range(2 * PACK):
            k, half = j >> 1, j & 1
            srcf = highf if half else lowf
            mk = (offt == j).astype(f32)
            sel = sel + srcf[k * EMB_DIM:(k + 1) * EMB_DIM, :] * mk
        et_ref[t * EMB_DIM:(t + 1) * EMB_DIM, :] = sel
    et_ref[N_TABLES * EMB_DIM:, :] = x32

    row = 0
    for i in range(1, N_FEAT):
        ei = et_ref[i * EMB_DIM:(i + 1) * EMB_DIM, :]
        for j in range(i):
            p = ei * et_ref[j * EMB_DIM:(j + 1) * EMB_DIM, :]
            inter_ref[row, :] = jnp.sum(p, axis=0)
            row += 1

    inter = inter_ref[...]
    z = jnp.dot(tw0a_ref[...], x32, preferred_element_type=f32)
    z = z + jnp.dot(tw0b_ref[...], inter, preferred_element_type=f32)
    z = jnp.maximum(z + tb0_ref[...], 0.0)
    z = jnp.maximum(jnp.dot(tw1_ref[...], z, preferred_element_type=f32)
                    + tb1_ref[...], 0.0)
    z = jnp.maximum(jnp.dot(tw2_ref[...], z, preferred_element_type=f32)
                    + tb2_ref[...], 0.0)
    z = jnp.maximum(jnp.dot(tw3_ref[...], z, preferred_element_type=f32)
                    + tb3_ref[...], 0.0)
    out_ref[...] = jnp.dot(tw4_ref[...], z, preferred_element_type=f32) + tb4_ref[...]


def _tc_forward(xt, g, off, bw0t, bb0, bw1t, bb1, bw2t, bb2,
                tw0at, tw0bt, tb0, tw1t, tb1, tw2t, tb2, tw3t, tb3, tw4t, tb4):
    full = lambda a: pl.BlockSpec(a.shape, lambda i: (0,) * a.ndim)
    weights = (bw0t, bb0, bw1t, bb1, bw2t, bb2,
               tw0at, tw0bt, tb0, tw1t, tb1, tw2t, tb2, tw3t, tb3, tw4t, tb4)
    return pl.pallas_call(
        _tc_body,
        grid=(_NB,),
        in_specs=[
            pl.BlockSpec((N_DENSE, _BB), lambda i: (0, i)),
            pl.BlockSpec((N_TABLES * _BB, 128), lambda i: (i, 0)),
            pl.BlockSpec((1, N_TABLES, _BB), lambda i: (i, 0, 0)),
        ] + [full(w) for w in weights],
        out_specs=pl.BlockSpec((1, _BB), lambda i: (0, i)),
        out_shape=jax.ShapeDtypeStruct((1, B), jnp.float32),
        scratch_shapes=[
            pltpu.VMEM((N_FEAT * EMB_DIM, _BB), jnp.float32),
            pltpu.VMEM((N_INTERACT, _BB), jnp.float32),
        ],
        compiler_params=pltpu.CompilerParams(
            dimension_semantics=("parallel",)),
    )(xt, g, off, *weights)


def kernel(numerical_features, cat_features, emb_tables,
           bW0, bb0, bW1, bb1, bW2, bb2,
           tW0, tb0, tW1, tb1, tW2, tb2, tW3, tb3, tW4, tb4):
    cat = cat_features.astype(jnp.int32)
    toff = jnp.arange(N_TABLES, dtype=jnp.int32)[:, None]
    q = cat % PIECE
    s_ = q // _RROWS
    r_ = q % _RROWS
    main_r = (toff * _RSTEPS + s_) * _HROWS + r_ % _HROWS
    main_k = (cat // PIECE) * 2 + r_ // _HROWS
    cp = cat - PACK * PIECE
    rt = jnp.where(cp >= 128, 32 + (cp - 128), cp & 31)
    kt = jnp.where(cp >= 128, 0, cp >> 5)
    tail_base = I32_MAIN_ROWS + toff * I32_TAIL_PER_TABLE
    tail_r = jnp.where(rt >= 32,
                       tail_base + 16 + (rt - 32) % 16,
                       tail_base + rt % 16)
    tail_half = jnp.where(rt >= 32, (rt - 32) // 16, rt // 16)
    tail_k = kt * 2 + tail_half
    is_tail = cp >= 0
    rows = jnp.where(is_tail, tail_r, main_r)
    offk = jnp.where(is_tail, tail_k, main_k)
    g_idx = rows.reshape(N_TABLES, _NB, _BB).transpose(1, 0, 2).reshape(-1)
    off = offk.reshape(N_TABLES, _NB, _BB).transpose(1, 0, 2)

    tables_dm = jnp.swapaxes(emb_tables, 1, 2)
    table128 = _repack(tables_dm)
    gathered = _sc_gather(table128, g_idx)

    col = lambda v: v.reshape(-1, 1)
    out = _tc_forward(
        numerical_features.T, gathered, off,
        bW0.T, col(bb0), bW1.T, col(bb1), bW2.T, col(bb2),
        tW0[:EMB_DIM].T, tW0[EMB_DIM:].T, col(tb0),
        tW1.T, col(tb1), tW2.T, col(tb2), tW3.T, col(tb3), tW4.T, col(tb4),
    )
    return out.T


# speedup vs baseline: 8.8935x; 1.0082x over previous
import functools

import jax
import jax.numpy as jnp
from jax import lax
from jax.experimental import pallas as pl
from jax.experimental.pallas import tpu as pltpu
from jax.experimental.pallas import tpu_sc as plsc

B = 4096
N_DENSE = 13
N_TABLES = 26
VOCAB = 100000
EMB_DIM = 32
N_FEAT = N_TABLES + 1
N_INTERACT = N_FEAT * (N_FEAT - 1) // 2
PACK = 4
PIECE = 24960
MAIN_ROWS = N_TABLES * PIECE
TAIL_ROWS_PER_TABLE = 64
NPACKED = MAIN_ROWS + N_TABLES * TAIL_ROWS_PER_TABLE
I32_MAIN_ROWS = MAIN_ROWS // 2
I32_TAIL_PER_TABLE = TAIL_ROWS_PER_TABLE // 2
NPACKED_I32 = I32_MAIN_ROWS + N_TABLES * I32_TAIL_PER_TABLE

_SC_NUM_CORES = 2
_SC_NUM_SUBCORES = 16
_NW = _SC_NUM_CORES * _SC_NUM_SUBCORES
_CHUNK = 416

_BB = 512
_NB = B // _BB


_RSTEPS = 3
_RROWS = PIECE // _RSTEPS
_HROWS = _RROWS // 2


def _pack_bf16_pair(lo_f32_bits, hi_f32_bits):
    lo = jnp.bitwise_and(jnp.right_shift(lo_f32_bits + 32768, 16),
                         jnp.int32(0xFFFF))
    hi = jnp.bitwise_and(hi_f32_bits + 32768, jnp.int32(-65536))
    return jnp.bitwise_or(hi, lo)


def _repack_main_body(q0_ref, q1_ref, q2_ref, q3_ref, out_ref):
    xcat = jnp.concatenate(
        [q0_ref[0], q1_ref[0], q2_ref[0], q3_ref[0]], axis=0)
    y = jax.lax.bitcast_convert_type(jnp.transpose(xcat), jnp.int32)
    out_ref[...] = _pack_bf16_pair(y[:_HROWS, :], y[_HROWS:, :])


def _repack_tail_body(src_hbm, prev_ref, out_ref, buf0, buf1, sems):
    del prev_ref
    cps = []
    for t in range(N_TABLES):
        cp0 = pltpu.make_async_copy(
            src_hbm.at[t, :, pl.ds(PACK * PIECE, 128)], buf0.at[t], sems.at[t, 0])
        cp1 = pltpu.make_async_copy(
            src_hbm.at[t, :, pl.ds(PACK * PIECE + 128, 32)], buf1.at[t], sems.at[t, 1])
        cp0.start()
        cp1.start()
        cps.extend((cp0, cp1))
    for cp in cps:
        cp.wait()
    pad = jnp.zeros((32, 128 - EMB_DIM), dtype=jnp.float32)
    for t in range(N_TABLES):
        t0 = jnp.transpose(buf0[t])
        w0 = jnp.concatenate(
            [t0[k * 32:(k + 1) * 32, :] for k in range(PACK)], axis=1)
        w0 = jax.lax.bitcast_convert_type(w0, jnp.int32)
        out_ref[32 * t:32 * t + 16, :] = _pack_bf16_pair(w0[0:16, :], w0[16:32, :])
        t1 = jnp.transpose(buf1[t])
        w1 = jax.lax.bitcast_convert_type(
            jnp.concatenate([t1, pad], axis=1), jnp.int32)
        out_ref[32 * t + 16:32 * t + 32, :] = _pack_bf16_pair(w1[0:16, :], w1[16:32, :])


def _repack(tables_dm):
    def qspec(k):
        return pl.BlockSpec((1, EMB_DIM, _RROWS),
                            lambda t, s, _k=k: (t, 0, _k * _RSTEPS + s))

    main = pl.pallas_call(
        _repack_main_body,
        grid=(N_TABLES, _RSTEPS),
        in_specs=[qspec(k) for k in range(PACK)],
        out_specs=pl.BlockSpec((_HROWS, PACK * EMB_DIM),
                               lambda t, s: (t * _RSTEPS + s, 0)),
        out_shape=jax.ShapeDtypeStruct((NPACKED_I32, PACK * EMB_DIM), jnp.int32),
        compiler_params=pltpu.CompilerParams(
            dimension_semantics=("parallel", "parallel")),
    )(tables_dm, tables_dm, tables_dm, tables_dm)
    return pl.pallas_call(
        _repack_tail_body,
        grid=(1,),
        in_specs=[pl.BlockSpec(memory_space=pl.ANY),
                  pl.BlockSpec(memory_space=pl.ANY)],
        out_specs=pl.BlockSpec((N_TABLES * I32_TAIL_PER_TABLE, PACK * EMB_DIM),
                               lambda i: (I32_MAIN_ROWS //
                                          (N_TABLES * I32_TAIL_PER_TABLE), 0)),
        out_shape=jax.ShapeDtypeStruct((NPACKED_I32, PACK * EMB_DIM), jnp.int32),
        input_output_aliases={1: 0},
        scratch_shapes=[
            pltpu.VMEM((N_TABLES, EMB_DIM, 128), jnp.float32),
            pltpu.VMEM((N_TABLES, EMB_DIM, 32), jnp.float32),
            pltpu.SemaphoreType.DMA((N_TABLES, 2)),
        ],
    )(tables_dm, main)


def _sc_gather(table128, idx_flat):
    ni = idx_flat.shape[0]
    b_per_w = ni // _NW
    n_chunks = b_per_w // _CHUNK
    mesh = plsc.VectorSubcoreMesh(core_axis_name="c", subcore_axis_name="s")

    @functools.partial(
        pl.kernel,
        mesh=mesh,
        out_type=jax.ShapeDtypeStruct((ni, 128), jnp.int32),
        scratch_types=[
            pltpu.VMEM((_CHUNK,), jnp.int32),
            pltpu.VMEM((_CHUNK,), jnp.int32),
            pltpu.VMEM((_CHUNK, 128), jnp.int32),
            pltpu.VMEM((_CHUNK, 128), jnp.int32),
            pltpu.SemaphoreType.DMA((2,)),
        ],
    )
    def gather_kernel(table_hbm, idx_hbm, out_hbm, idx_a, idx_b, rows_a,
                      rows_b, sems):
        wid = lax.axis_index("s") * _SC_NUM_CORES + lax.axis_index("c")
        wbase = wid * b_per_w
        idx_bufs = (idx_a, idx_b)
        row_bufs = (rows_a, rows_b)

        pltpu.sync_copy(idx_hbm.at[pl.ds(wbase, _CHUNK)], idx_a)
        cps = [pltpu.async_copy(table_hbm.at[idx_a], rows_a, sems.at[0])]
        for c in range(n_chunks):
            cps[c].wait()
            if c + 1 < n_chunks:
                nxt = (c + 1) % 2
                pltpu.sync_copy(
                    idx_hbm.at[pl.ds(wbase + (c + 1) * _CHUNK, _CHUNK)],
                    idx_bufs[nxt])
                cps.append(pltpu.async_copy(
                    table_hbm.at[idx_bufs[nxt]], row_bufs[nxt], sems.at[nxt]))
            pltpu.sync_copy(row_bufs[c % 2],
                            out_hbm.at[pl.ds(wbase + c * _CHUNK, _CHUNK)])

    return gather_kernel(table128, idx_flat)


def _tc_body(xt_ref, g_ref, off_ref,
             bw0_ref, bb0_ref, bw1_ref, bb1_ref, bw2_ref, bb2_ref,
             tw0a_ref, tw0b_ref, tb0_ref, tw1_ref, tb1_ref,
             tw2_ref, tb2_ref, tw3_ref, tb3_ref, tw4_ref, tb4_ref,
             out_ref, et_ref, inter_ref):
    f32 = jnp.float32
    x = xt_ref[...]
    h = jnp.maximum(jnp.dot(bw0_ref[...], x, preferred_element_type=f32)
                    + bb0_ref[...], 0.0)
    h = jnp.maximum(jnp.dot(bw1_ref[...], h, preferred_element_type=f32)
                    + bb1_ref[...], 0.0)
    x32 = jnp.maximum(jnp.dot(bw2_ref[...], h, preferred_element_type=f32)
                      + bb2_ref[...], 0.0)

    for t in range(N_TABLES):
        sti = jnp.transpose(g_ref[t * _BB:(t + 1) * _BB, :])
        lowf = jax.lax.bitcast_convert_type(
            jnp.left_shift(sti, 16), f32)
        highf = jax.lax.bitcast_convert_type(
            jnp.bitwise_and(sti, jnp.int32(-65536)), f32)
        offt = off_ref[0, t:t + 1, :]
        sel = jnp.zeros((EMB_DIM, _BB), dtype=f32)
        for j in range(2 * PACK):
            k, half = j >> 1, j & 1
            srcf = highf if half else lowf
            mk = (offt == j).astype(f32)
            sel = sel + srcf[k * EMB_DIM:(k + 1) * EMB_DIM, :] * mk
        et_ref[t * EMB_DIM:(t + 1) * EMB_DIM, :] = sel
    et_ref[N_TABLES * EMB_DIM:, :] = x32

    row = 0
    for i in range(1, N_FEAT):
        ei = et_ref[i * EMB_DIM:(i + 1) * EMB_DIM, :]
        for j in range(i):
            p = ei * et_ref[j * EMB_DIM:(j + 1) * EMB_DIM, :]
            inter_ref[row, :] = jnp.sum(p, axis=0)
            row += 1

    inter = inter_ref[...]
    z = jnp.dot(tw0a_ref[...], x32, preferred_element_type=f32)
    z = z + jnp.dot(tw0b_ref[...], inter, preferred_element_type=f32)
    z = jnp.maximum(z + tb0_ref[...], 0.0)
    z = jnp.maximum(jnp.dot(tw1_ref[...], z, preferred_element_type=f32)
                    + tb1_ref[...], 0.0)
    z = jnp.maximum(jnp.dot(tw2_ref[...], z, preferred_element_type=f32)
                    + tb2_ref[...], 0.0)
    z = jnp.maximum(jnp.dot(tw3_ref[...], z, preferred_element_type=f32)
                    + tb3_ref[...], 0.0)
    out_ref[...] = jnp.dot(tw4_ref[...], z, preferred_element_type=f32) + tb4_ref[...]


def _tc_forward(xt, g, off, bw0t, bb0, bw1t, bb1, bw2t, bb2,
                tw0at, tw0bt, tb0, tw1t, tb1, tw2t, tb2, tw3t, tb3, tw4t, tb4):
    full = lambda a: pl.BlockSpec(a.shape, lambda i: (0,) * a.ndim)
    weights = (bw0t, bb0, bw1t, bb1, bw2t, bb2,
               tw0at, tw0bt, tb0, tw1t, tb1, tw2t, tb2, tw3t, tb3, tw4t, tb4)
    return pl.pallas_call(
        _tc_body,
        grid=(_NB,),
        in_specs=[
            pl.BlockSpec((N_DENSE, _BB), lambda i: (0, i)),
            pl.BlockSpec((N_TABLES * _BB, 128), lambda i: (i, 0)),
            pl.BlockSpec((1, N_TABLES, _BB), lambda i: (i, 0, 0)),
        ] + [full(w) for w in weights],
        out_specs=pl.BlockSpec((1, _BB), lambda i: (0, i)),
        out_shape=jax.ShapeDtypeStruct((1, B), jnp.float32),
        scratch_shapes=[
            pltpu.VMEM((N_FEAT * EMB_DIM, _BB), jnp.float32),
            pltpu.VMEM((N_INTERACT, _BB), jnp.float32),
        ],
        compiler_params=pltpu.CompilerParams(
            dimension_semantics=("parallel",)),
    )(xt, g, off, *weights)


def kernel(numerical_features, cat_features, emb_tables,
           bW0, bb0, bW1, bb1, bW2, bb2,
           tW0, tb0, tW1, tb1, tW2, tb2, tW3, tb3, tW4, tb4):
    cat = cat_features.astype(jnp.int32)
    toff = jnp.arange(N_TABLES, dtype=jnp.int32)[:, None]
    q = cat % PIECE
    s_ = q // _RROWS
    r_ = q % _RROWS
    main_r = (toff * _RSTEPS + s_) * _HROWS + r_ % _HROWS
    main_k = (cat // PIECE) * 2 + r_ // _HROWS
    cp = cat - PACK * PIECE
    rt = jnp.where(cp >= 128, 32 + (cp - 128), cp & 31)
    kt = jnp.where(cp >= 128, 0, cp >> 5)
    tail_base = I32_MAIN_ROWS + toff * I32_TAIL_PER_TABLE
    tail_r = jnp.where(rt >= 32,
                       tail_base + 16 + (rt - 32) % 16,
                       tail_base + rt % 16)
    tail_half = jnp.where(rt >= 32, (rt - 32) // 16, rt // 16)
    tail_k = kt * 2 + tail_half
    is_tail = cp >= 0
    rows = jnp.where(is_tail, tail_r, main_r)
    offk = jnp.where(is_tail, tail_k, main_k)
    g_idx = rows.reshape(N_TABLES, _NB, _BB).transpose(1, 0, 2).reshape(-1)
    off = offk.reshape(N_TABLES, _NB, _BB).transpose(1, 0, 2)

    tables_dm = jnp.swapaxes(emb_tables, 1, 2)
    table128 = _repack(tables_dm)
    gathered = _sc_gather(table128, g_idx)

    col = lambda v: v.reshape(-1, 1)
    out = _tc_forward(
        numerical_features.T, gathered, off,
        bW0.T, col(bb0), bW1.T, col(bb1), bW2.T, col(bb2),
        tW0[:EMB_DIM].T, tW0[EMB_DIM:].T, col(tb0),
        tW1.T, col(tb1), tW2.T, col(tb2), tW3.T, col(tb3), tW4.T, col(tb4),
    )
    return out.T
